# baseline (speedup 1.0000x reference)
import jax
import jax.numpy as jnp
from jax.experimental import pallas as pl
from jax.experimental.pallas import tpu as pltpu

_ROWS = 8192
_C = _ROWS // 128


def _body(uf_ref, d1f_ref, d2f_ref, v1_ref, v2_ref, o_ref, s1t_ref, s2t_ref):
    i = pl.program_id(0)

    @pl.when(i == 0)
    def _():
        uf = uf_ref[...]
        m1 = jnp.max(uf)
        m2 = jnp.max(uf - d2f_ref[...])
        s1t_ref[...] = jnp.transpose(jnp.minimum(d1f_ref[...], m2))
        s2t_ref[...] = jnp.transpose(jnp.minimum(d2f_ref[...], m1))

    ones_row = jnp.ones((1, 128), jnp.float32)
    for step in range(2):

        @pl.when(i == step)
        def _():
            for k in range(_C):
                col = step * _C + k
                sl = slice(128 * k, 128 * (k + 1))
                s1b = jax.lax.dot(s1t_ref[:, col : col + 1], ones_row)
                s2b = jax.lax.dot(s2t_ref[:, col : col + 1], ones_row)
                o_ref[sl, :] = v1_ref[sl, :] * s1b + v2_ref[sl, :] * s2b


def kernel(u, d1, d2, v1, v2):
    B, D = v1.shape
    uf = u.reshape(B // 128, 128)
    d1f = d1.reshape(B // 128, 128)
    d2f = d2.reshape(B // 128, 128)
    grid = (B // _ROWS,)
    out = pl.pallas_call(
        _body,
        grid=grid,
        in_specs=[
            pl.BlockSpec((B // 128, 128), lambda i: (0, 0)),
            pl.BlockSpec((B // 128, 128), lambda i: (0, 0)),
            pl.BlockSpec((B // 128, 128), lambda i: (0, 0)),
            pl.BlockSpec((_ROWS, D), lambda i: (i, 0)),
            pl.BlockSpec((_ROWS, D), lambda i: (i, 0)),
        ],
        out_specs=pl.BlockSpec((_ROWS, D), lambda i: (i, 0)),
        out_shape=jax.ShapeDtypeStruct((B, D), jnp.float32),
        scratch_shapes=[
            pltpu.VMEM((128, 128), jnp.float32),
            pltpu.VMEM((128, 128), jnp.float32),
        ],
    )(uf, d1f, d2f, v1, v2)
    return out


# speedup vs baseline: 1.0125x; 1.0125x over previous
import jax
import jax.numpy as jnp
from jax.experimental import pallas as pl
from jax.experimental.pallas import tpu as pltpu

_ROWS = 8192
_C = _ROWS // 128


def _body(uf_ref, d1f_ref, d2f_ref, v1_ref, v2_ref, o_ref, s1t_ref, s2t_ref):
    i = pl.program_id(0)

    @pl.when(i == 0)
    def _():
        uf = uf_ref[...]
        m1 = jnp.max(uf)
        m2 = jnp.max(uf - d2f_ref[...])
        s1t_ref[...] = jnp.transpose(jnp.minimum(d1f_ref[...], m2))
        s2t_ref[...] = jnp.transpose(jnp.minimum(d2f_ref[...], m1))

    ones_row = jnp.ones((1, 128), jnp.float32)
    for step in range(2):

        @pl.when(i == step)
        def _():
            for k in range(_C):
                col = step * _C + k
                sl = slice(128 * k, 128 * (k + 1))
                o_ref[sl, :] = (
                    v1_ref[sl, :] * s1t_ref[:, col : col + 1]
                    + v2_ref[sl, :] * s2t_ref[:, col : col + 1]
                )


def kernel(u, d1, d2, v1, v2):
    B, D = v1.shape
    uf = u.reshape(B // 128, 128)
    d1f = d1.reshape(B // 128, 128)
    d2f = d2.reshape(B // 128, 128)
    grid = (B // _ROWS,)
    out = pl.pallas_call(
        _body,
        grid=grid,
        in_specs=[
            pl.BlockSpec((B // 128, 128), lambda i: (0, 0)),
            pl.BlockSpec((B // 128, 128), lambda i: (0, 0)),
            pl.BlockSpec((B // 128, 128), lambda i: (0, 0)),
            pl.BlockSpec((_ROWS, D), lambda i: (i, 0)),
            pl.BlockSpec((_ROWS, D), lambda i: (i, 0)),
        ],
        out_specs=pl.BlockSpec((_ROWS, D), lambda i: (i, 0)),
        out_shape=jax.ShapeDtypeStruct((B, D), jnp.float32),
        scratch_shapes=[
            pltpu.VMEM((128, 128), jnp.float32),
            pltpu.VMEM((128, 128), jnp.float32),
        ],
    )(uf, d1f, d2f, v1, v2)
    return out
